# baseline (speedup 1.0000x reference)
import jax
import jax.numpy as jnp
from jax import lax
from jax.experimental import pallas as pl
from jax.experimental.pallas import tpu as pltpu
from jax.experimental.pallas import tpu_sc as plsc

NUM_FIELDS = 26
FIELD_DIM = 100000
TOTAL_ROWS = NUM_FIELDS * FIELD_DIM
BATCH = 16384
L = 16
NC, NS = 2, 16
NW = NC * NS
B_PER_W = BATCH // NW
N_PER_W = B_PER_W * NUM_FIELDS
CHUNK = 128
UNROLL = 4
GROUP_FIELDS = 2
N_GROUPS = NUM_FIELDS // GROUP_FIELDS
N_PER_G = GROUP_FIELDS * B_PER_W
CHUNKS_PER_G = N_PER_G // CHUNK
SPLIT = 7


def _index_body(xt_hbm, idx_hbm, idx_v, sem):
    wid = lax.axis_index("c") * NS + lax.axis_index("s")

    for f in range(NUM_FIELDS):
        pltpu.make_async_copy(
            xt_hbm.at[pl.ds(f * BATCH + wid * B_PER_W, B_PER_W)],
            idx_v.at[pl.ds(f * B_PER_W, B_PER_W)],
            sem,
        ).start()
    for f in range(NUM_FIELDS):
        pltpu.make_async_copy(
            xt_hbm.at[pl.ds(f * BATCH + wid * B_PER_W, B_PER_W)],
            idx_v.at[pl.ds(f * B_PER_W, B_PER_W)],
            sem,
        ).wait()

    def add_offsets(f, carry):
        off = (f % GROUP_FIELDS) * FIELD_DIM

        def inner(c, carry2):
            for u in range(UNROLL):
                o = f * B_PER_W + (c * UNROLL + u) * L
                idx_v[pl.ds(o, L)] = idx_v[pl.ds(o, L)] + off
            return carry2

        return lax.fori_loop(0, B_PER_W // (L * UNROLL), inner, carry)

    lax.fori_loop(0, NUM_FIELDS, add_offsets, 0)

    pltpu.sync_copy(idx_v, idx_hbm.at[pl.ds(wid * N_PER_W, N_PER_W)])


def _make_gather_body(g0, ng):
    n_local = ng * N_PER_G

    def body(idx_hbm, *refs):
        tabs = refs[:ng]
        prev_hbm, out_hbm, idx_v, rows_v, out_v, prev_v, sem = refs[ng:]
        wid = lax.axis_index("c") * NS + lax.axis_index("s")

        pltpu.sync_copy(
            idx_hbm.at[pl.ds(wid * N_PER_W + g0 * N_PER_G, n_local)], idx_v
        )
        pltpu.sync_copy(prev_hbm.at[pl.ds(wid * B_PER_W, B_PER_W)], prev_v)

        for g in range(ng):
            def fire(j, carry, g=g):
                o = g * N_PER_G + j * CHUNK
                pltpu.make_async_copy(
                    tabs[g].at[idx_v.at[pl.ds(o, CHUNK)]],
                    rows_v.at[pl.ds(o, CHUNK)],
                    sem,
                ).start()
                return carry

            lax.fori_loop(0, CHUNKS_PER_G, fire, 0)

        for g in range(ng):
            def drain(j, carry, g=g):
                o = g * N_PER_G + j * CHUNK
                pltpu.make_async_copy(
                    tabs[g].at[idx_v.at[pl.ds(o, CHUNK)]],
                    rows_v.at[pl.ds(o, CHUNK)],
                    sem,
                ).wait()
                return carry

            lax.fori_loop(0, CHUNKS_PER_G, drain, 0)

        def reduce(c, carry):
            o = c * L
            acc = prev_v[pl.ds(o, L)]
            for f in range(ng * GROUP_FIELDS):
                acc = acc + rows_v[pl.ds(f * B_PER_W + o, L)]
            out_v[pl.ds(o, L)] = acc
            return carry

        lax.fori_loop(0, B_PER_W // L, reduce, 0)

        pltpu.sync_copy(out_v, out_hbm.at[pl.ds(wid * B_PER_W, B_PER_W)])

    return body


def _gather_call(g0, ng, idx, tabs, prev):
    mesh = plsc.VectorSubcoreMesh(core_axis_name="c", subcore_axis_name="s")
    n_local = ng * N_PER_G
    return pl.kernel(
        _make_gather_body(g0, ng),
        out_type=jax.ShapeDtypeStruct((BATCH,), jnp.float32),
        mesh=mesh,
        scratch_types=[
            pltpu.VMEM((n_local,), jnp.int32),
            pltpu.VMEM((n_local,), jnp.float32),
            pltpu.VMEM((B_PER_W,), jnp.float32),
            pltpu.VMEM((B_PER_W,), jnp.float32),
            pltpu.SemaphoreType.DMA,
        ],
    )(idx, *tabs, prev)


@jax.jit
def _run(xt_flat, tabs, bias_b):
    mesh = plsc.VectorSubcoreMesh(core_axis_name="c", subcore_axis_name="s")
    idx = pl.kernel(
        _index_body,
        out_type=jax.ShapeDtypeStruct((BATCH * NUM_FIELDS,), jnp.int32),
        mesh=mesh,
        scratch_types=[
            pltpu.VMEM((N_PER_W,), jnp.int32),
            pltpu.SemaphoreType.DMA,
        ],
    )(xt_flat)
    part = _gather_call(0, SPLIT, idx, tabs[:SPLIT], bias_b)
    return _gather_call(SPLIT, N_GROUPS - SPLIT, idx, tabs[SPLIT:], part)


def kernel(x, table, bias):
    xt_flat = x.T.reshape(-1)
    tabs = tuple(
        table[g * GROUP_FIELDS * FIELD_DIM:(g + 1) * GROUP_FIELDS * FIELD_DIM].reshape(-1)
        for g in range(N_GROUPS)
    )
    bias_b = jnp.broadcast_to(bias, (BATCH,))
    out = _run(xt_flat, tabs, bias_b)
    return out.reshape(BATCH, 1)


# speedup vs baseline: 1.0778x; 1.0778x over previous
import jax
import jax.numpy as jnp
from jax import lax
from jax.experimental import pallas as pl
from jax.experimental.pallas import tpu as pltpu
from jax.experimental.pallas import tpu_sc as plsc

NUM_FIELDS = 26
FIELD_DIM = 100000
TOTAL_ROWS = NUM_FIELDS * FIELD_DIM
BATCH = 16384
L = 16
NC, NS = 2, 16
NW = NC * NS
B_PER_W = BATCH // NW
N_PER_W = B_PER_W * NUM_FIELDS
CHUNK = 128
N_CHUNKS = N_PER_W // CHUNK
UNROLL = 4
GROUP_FIELDS = 2
N_GROUPS = NUM_FIELDS // GROUP_FIELDS
N_PER_G = GROUP_FIELDS * B_PER_W
CHUNKS_PER_G = N_PER_G // CHUNK


def _index_body(xt_hbm, idx_hbm, idx_v, sem):
    wid = lax.axis_index("c") * NS + lax.axis_index("s")

    for f in range(NUM_FIELDS):
        pltpu.make_async_copy(
            xt_hbm.at[pl.ds(f * BATCH + wid * B_PER_W, B_PER_W)],
            idx_v.at[pl.ds(f * B_PER_W, B_PER_W)],
            sem,
        ).start()
    for f in range(NUM_FIELDS):
        pltpu.make_async_copy(
            xt_hbm.at[pl.ds(f * BATCH + wid * B_PER_W, B_PER_W)],
            idx_v.at[pl.ds(f * B_PER_W, B_PER_W)],
            sem,
        ).wait()

    def add_offsets(f, carry):
        off = (f % GROUP_FIELDS) * FIELD_DIM

        def inner(c, carry2):
            for u in range(UNROLL):
                o = f * B_PER_W + (c * UNROLL + u) * L
                idx_v[pl.ds(o, L)] = idx_v[pl.ds(o, L)] + off
            return carry2

        return lax.fori_loop(0, B_PER_W // (L * UNROLL), inner, carry)

    lax.fori_loop(0, NUM_FIELDS, add_offsets, 0)

    pltpu.sync_copy(idx_v, idx_hbm.at[pl.ds(wid * N_PER_W, N_PER_W)])


def _gather_body(idx_hbm, *refs):
    tabs = refs[:N_GROUPS]
    bias_hbm, out_hbm, idx_v, rows_v, out_v, bias_v, sem = refs[N_GROUPS:]
    wid = lax.axis_index("c") * NS + lax.axis_index("s")

    pltpu.sync_copy(idx_hbm.at[pl.ds(wid * N_PER_W, N_PER_W)], idx_v)
    pltpu.sync_copy(bias_hbm, bias_v)

    for g in range(N_GROUPS):
        def fire(j, carry, g=g):
            o = g * N_PER_G + j * CHUNK
            pltpu.make_async_copy(
                tabs[g].at[idx_v.at[pl.ds(o, CHUNK)]],
                rows_v.at[pl.ds(o, CHUNK)],
                sem,
            ).start()
            return carry

        lax.fori_loop(0, CHUNKS_PER_G, fire, 0)

    for g in range(N_GROUPS):
        def drain(j, carry, g=g):
            o = g * N_PER_G + j * CHUNK
            pltpu.make_async_copy(
                tabs[g].at[idx_v.at[pl.ds(o, CHUNK)]],
                rows_v.at[pl.ds(o, CHUNK)],
                sem,
            ).wait()
            return carry

        lax.fori_loop(0, CHUNKS_PER_G, drain, 0)

    def reduce(c, carry):
        o = c * L
        acc = bias_v[...]
        for f in range(NUM_FIELDS):
            acc = acc + rows_v[pl.ds(f * B_PER_W + o, L)]
        out_v[pl.ds(o, L)] = acc
        return carry

    lax.fori_loop(0, B_PER_W // L, reduce, 0)

    pltpu.sync_copy(out_v, out_hbm.at[pl.ds(wid * B_PER_W, B_PER_W)])


@jax.jit
def _run(xt_flat, tabs, bias16):
    mesh = plsc.VectorSubcoreMesh(core_axis_name="c", subcore_axis_name="s")
    idx = pl.kernel(
        _index_body,
        out_type=jax.ShapeDtypeStruct((BATCH * NUM_FIELDS,), jnp.int32),
        mesh=mesh,
        scratch_types=[
            pltpu.VMEM((N_PER_W,), jnp.int32),
            pltpu.SemaphoreType.DMA,
        ],
    )(xt_flat)
    return pl.kernel(
        _gather_body,
        out_type=jax.ShapeDtypeStruct((BATCH,), jnp.float32),
        mesh=mesh,
        scratch_types=[
            pltpu.VMEM((N_PER_W,), jnp.int32),
            pltpu.VMEM((N_PER_W,), jnp.float32),
            pltpu.VMEM((B_PER_W,), jnp.float32),
            pltpu.VMEM((L,), jnp.float32),
            pltpu.SemaphoreType.DMA,
        ],
    )(idx, *tabs, bias16)


def kernel(x, table, bias):
    xt_flat = x.T.reshape(-1)
    tabs = tuple(
        table[g * GROUP_FIELDS * FIELD_DIM:(g + 1) * GROUP_FIELDS * FIELD_DIM].reshape(-1)
        for g in range(N_GROUPS)
    )
    bias16 = jnp.broadcast_to(bias, (L,))
    out = _run(xt_flat, tabs, bias16)
    return out.reshape(BATCH, 1)
